# baseline (speedup 1.0000x reference)
import functools

import jax
import jax.numpy as jnp
from jax import lax
from jax.experimental import pallas as pl
from jax.experimental.pallas import tpu as pltpu
from jax.experimental.pallas import tpu_sc as plsc

BATCH = 4096
SEQ = 200
D = 64
SCALE = 8.0

_info = plsc.get_sparse_core_info()
NC, NS, L = _info.num_cores, _info.num_subcores, _info.num_lanes
NW = NC * NS

B_TOTAL = BATCH * SEQ
B_PER_W = B_TOTAL // NW
CB = 128
N_CHUNKS = B_PER_W // CB


def _emb_body(x_hbm, table_hbm, out_hbm, idx_v, rows_v, sem):
    wid = lax.axis_index("s") * NC + lax.axis_index("c")
    base = wid * B_PER_W

    def chunk(j, carry):
        off = base + j * CB
        pltpu.sync_copy(x_hbm.at[pl.ds(off, CB)], idx_v)
        pltpu.async_copy(table_hbm.at[idx_v], rows_v, sem).wait()

        def row(r, c2):
            for c in range(D // L):
                rows_v[r, pl.ds(c * L, L)] = rows_v[r, pl.ds(c * L, L)] * SCALE
            return c2

        lax.fori_loop(0, CB, row, 0)
        pltpu.sync_copy(rows_v, out_hbm.at[pl.ds(off, CB)])
        return carry

    lax.fori_loop(0, N_CHUNKS, chunk, 0)


_emb_kernel = functools.partial(
    pl.kernel,
    out_type=jax.ShapeDtypeStruct((B_TOTAL, D), jnp.float32),
    mesh=plsc.VectorSubcoreMesh(core_axis_name="c", subcore_axis_name="s"),
    compiler_params=pltpu.CompilerParams(use_tc_tiling_on_sc=False),
    scratch_types=[
        pltpu.VMEM((CB,), jnp.int32),
        pltpu.VMEM((CB, D), jnp.float32),
        pltpu.SemaphoreType.DMA,
    ],
)(_emb_body)


def kernel(x, table):
    xf = x.reshape(B_TOTAL)
    out = _emb_kernel(xf, table)
    return out.reshape(BATCH, SEQ, D)


# speedup vs baseline: 1.2751x; 1.2751x over previous
import functools

import jax
import jax.numpy as jnp
from jax import lax
from jax.experimental import pallas as pl
from jax.experimental.pallas import tpu as pltpu
from jax.experimental.pallas import tpu_sc as plsc

BATCH = 4096
SEQ = 200
D = 64
SCALE = 8.0

_info = plsc.get_sparse_core_info()
NC, NS, L = _info.num_cores, _info.num_subcores, _info.num_lanes
NW = NC * NS

B_TOTAL = BATCH * SEQ
B_PER_W = B_TOTAL // NW
CB = 128
N_CHUNKS = B_PER_W // CB
NB = 4
LA = 2


def _emb_body(x_hbm, table_hbm, out_hbm, idx_slab, rows, *sems):
    gsem, ssem = sems[:NB], sems[NB:]
    wid = lax.axis_index("s") * NC + lax.axis_index("c")
    rbase = wid * N_CHUNKS
    obase = wid * B_PER_W

    pltpu.sync_copy(x_hbm.at[pl.ds(rbase, N_CHUNKS)], idx_slab)

    def start_gather(j, b):
        pltpu.async_copy(table_hbm.at[idx_slab.at[j]], rows.at[b], gsem[b])

    def wait_gather(j, b):
        pltpu.make_async_copy(table_hbm.at[idx_slab.at[j]], rows.at[b],
                              gsem[b]).wait()

    def start_scatter(j, b):
        pltpu.async_copy(rows.at[b], out_hbm.at[pl.ds(obase + j * CB, CB)],
                         ssem[b])

    def wait_scatter(j, b):
        pltpu.make_async_copy(rows.at[b], out_hbm.at[pl.ds(obase + j * CB, CB)],
                              ssem[b]).wait()

    for b in range(LA):
        start_gather(b, b)

    def group(g, carry):
        for b in range(NB):
            j = g * NB + b
            jf = j + LA
            bf = (b + LA) % NB

            @pl.when(jnp.logical_and(jf < N_CHUNKS, jf >= NB))
            def _():
                wait_scatter(jf - NB, bf)

            @pl.when(jf < N_CHUNKS)
            def _():
                start_gather(jf, bf)

            wait_gather(j, b)

            def row(i, c2):
                r = i * 2
                for rr in range(2):
                    for c in range(D // L):
                        rows[b, r + rr, pl.ds(c * L, L)] = (
                            rows[b, r + rr, pl.ds(c * L, L)] * SCALE)
                return c2

            lax.fori_loop(0, CB // 2, row, 0)
            start_scatter(j, b)
        return carry

    lax.fori_loop(0, N_CHUNKS // NB, group, 0)

    for b in range(NB):
        wait_scatter(N_CHUNKS - NB + b, b)


_emb_kernel = functools.partial(
    pl.kernel,
    out_type=jax.ShapeDtypeStruct((B_TOTAL, D), jnp.float32),
    mesh=plsc.VectorSubcoreMesh(core_axis_name="c", subcore_axis_name="s"),
    compiler_params=pltpu.CompilerParams(use_tc_tiling_on_sc=False),
    scratch_types=(
        [pltpu.VMEM((N_CHUNKS, CB), jnp.int32),
         pltpu.VMEM((NB, CB, D), jnp.float32)]
        + [pltpu.SemaphoreType.DMA] * (2 * NB)
    ),
)(_emb_body)


def kernel(x, table):
    xf = x.reshape(B_TOTAL // CB, CB)
    out = _emb_kernel(xf, table)
    return out.reshape(BATCH, SEQ, D)
